# Initial kernel scaffold; baseline (speedup 1.0000x reference)
import jax
import jax.numpy as jnp
from jax.experimental import pallas as pl


def kernel(x, edge_index, edge_attr):
    raise NotImplementedError("write your pallas kernel here")


# speedup vs baseline: 4.8196x; 4.8196x over previous
import functools

import jax
import jax.numpy as jnp
from jax import lax
from jax.experimental import pallas as pl
from jax.experimental.pallas import tpu as pltpu
from jax.experimental.pallas import tpu_sc as plsc

N_NODES = 10000
N_EDGES = 320000
D_FEAT = 128
D_EDGE = 16
G = 128
GPW = 80
N_WORKERS = 32
NG = N_EDGES // G
NG_PAD = N_WORKERS * GPW
E_PAD = NG_PAD * G
N_TILES = 16
N_PAD = 10240
ROWS_PER_TILE = N_PAD // N_TILES
PACK = D_FEAT // D_EDGE

_mesh = plsc.VectorSubcoreMesh(core_axis_name="c", subcore_axis_name="s")


def _zero_rows(buf, n):
    zz = jnp.zeros((16,), jnp.float32)

    def zrow(i, carry):
        for c in range(D_FEAT // 16):
            buf[i, pl.ds(c * 16, 16)] = zz
        return carry

    lax.fori_loop(0, n, zrow, 0)


def _sc_scatter_x(x, src3d, dst3d):
    @functools.partial(
        pl.kernel,
        mesh=_mesh,
        out_type=jax.ShapeDtypeStruct((2, N_PAD, D_FEAT), jnp.float32),
        scratch_types=[
            pltpu.VMEM((1, G), jnp.int32),
            pltpu.VMEM((1, G), jnp.int32),
            pltpu.VMEM((G, D_FEAT), jnp.float32),
            pltpu.VMEM_SHARED((N_PAD, D_FEAT), jnp.float32),
            pltpu.SemaphoreType.DMA,
        ],
    )
    def k(x_hbm, src_hbm, dst_hbm, px_hbm, sidx, didx, rows, acc_x, sem):
        cid = lax.axis_index("c")
        sid = lax.axis_index("s")
        wid = sid * 2 + cid

        _zero_rows(rows, G)
        for i in range(ROWS_PER_TILE // G):
            b = sid * ROWS_PER_TILE + i * G
            pltpu.sync_copy(rows, acc_x.at[pl.ds(b, G)])
        plsc.subcore_barrier()

        def group(g, carry):
            r = wid * GPW + g
            pltpu.sync_copy(src_hbm.at[r], sidx)
            pltpu.sync_copy(dst_hbm.at[r], didx)
            pltpu.async_copy(x_hbm.at[sidx.at[0]], rows, sem).wait()
            pltpu.sync_copy(rows, acc_x.at[didx.at[0]], add=True)
            return carry

        lax.fori_loop(0, GPW, group, 0)
        plsc.subcore_barrier()

        b = sid * ROWS_PER_TILE
        pltpu.sync_copy(acc_x.at[pl.ds(b, ROWS_PER_TILE)],
                        px_hbm.at[cid].at[pl.ds(b, ROWS_PER_TILE)])

    return k(x, src3d, dst3d)


def _sc_scatter_e(ea3d, dst3d):
    @functools.partial(
        pl.kernel,
        mesh=_mesh,
        out_type=jax.ShapeDtypeStruct((2, N_PAD, D_FEAT), jnp.float32),
        scratch_types=[
            pltpu.VMEM((1, G), jnp.int32),
            pltpu.VMEM((D_EDGE, D_FEAT), jnp.float32),
            pltpu.VMEM((G, D_FEAT), jnp.float32),
            pltpu.VMEM_SHARED((N_PAD, D_FEAT), jnp.float32),
        ],
    )
    def k(ea_hbm, dst_hbm, pe_hbm, didx, packed, rows, acc_e):
        cid = lax.axis_index("c")
        sid = lax.axis_index("s")
        wid = sid * 2 + cid

        _zero_rows(rows, G)
        for i in range(ROWS_PER_TILE // G):
            b = sid * ROWS_PER_TILE + i * G
            pltpu.sync_copy(rows, acc_e.at[pl.ds(b, G)])
        plsc.subcore_barrier()

        def group(g, carry):
            r = wid * GPW + g
            pltpu.sync_copy(dst_hbm.at[r], didx)
            rc = jnp.minimum(r, NG - 1)
            pltpu.sync_copy(ea_hbm.at[rc], packed)
            for c in range(PACK):
                for a in range(D_EDGE):
                    rows[a * PACK + c, pl.ds(0, D_EDGE)] = (
                        packed[a, pl.ds(c * D_EDGE, D_EDGE)])
            pltpu.sync_copy(rows, acc_e.at[didx.at[0]], add=True)
            return carry

        lax.fori_loop(0, GPW, group, 0)
        plsc.subcore_barrier()

        b = sid * ROWS_PER_TILE
        pltpu.sync_copy(acc_e.at[pl.ds(b, ROWS_PER_TILE)],
                        pe_hbm.at[cid].at[pl.ds(b, ROWS_PER_TILE)])

    return k(ea3d, dst3d)


def _combine_body(pxa, pxb, pea, peb, o_ref):
    o_ref[:, :D_FEAT] = pxa[0] + pxb[0]
    o_ref[:, D_FEAT:] = pea[0][:, :D_EDGE] + peb[0][:, :D_EDGE]


def _combine(px, pe):
    B = 1000
    return pl.pallas_call(
        _combine_body,
        grid=(N_NODES // B,),
        in_specs=[
            pl.BlockSpec((1, B, D_FEAT), lambda i: (0, i, 0)),
            pl.BlockSpec((1, B, D_FEAT), lambda i: (1, i, 0)),
            pl.BlockSpec((1, B, D_FEAT), lambda i: (0, i, 0)),
            pl.BlockSpec((1, B, D_FEAT), lambda i: (1, i, 0)),
        ],
        out_specs=pl.BlockSpec((B, D_FEAT + D_EDGE), lambda i: (i, 0)),
        out_shape=jax.ShapeDtypeStruct((N_NODES, D_FEAT + D_EDGE), jnp.float32),
    )(px, px, pe, pe)


def kernel(x, edge_index, edge_attr):
    src = edge_index[0].astype(jnp.int32)
    dst = edge_index[1].astype(jnp.int32)
    pad = E_PAD - N_EDGES
    pad_ids = jnp.arange(pad, dtype=jnp.int32)
    src3d = jnp.concatenate(
        [src, pad_ids % N_NODES]).reshape(NG_PAD, 1, G)
    dst3d = jnp.concatenate(
        [dst, N_NODES + pad_ids % (N_PAD - N_NODES)]).reshape(NG_PAD, 1, G)
    ea3d = edge_attr.reshape(NG, D_EDGE, D_FEAT)
    px = _sc_scatter_x(x, src3d, dst3d)
    pe = _sc_scatter_e(ea3d, dst3d)
    return _combine(px, pe)


# speedup vs baseline: 5.9932x; 1.2435x over previous
import functools

import jax
import jax.numpy as jnp
from jax import lax
from jax.experimental import pallas as pl
from jax.experimental.pallas import tpu as pltpu
from jax.experimental.pallas import tpu_sc as plsc

N_NODES = 10000
N_EDGES = 320000
D_FEAT = 128
D_EDGE = 16
G = 128
GPW = 80
N_WORKERS = 32
NG = N_EDGES // G
NG_PAD = N_WORKERS * GPW
E_PAD = NG_PAD * G
N_TILES = 16
N_PAD = 10240
ROWS_PER_TILE = N_PAD // N_TILES
PACK = D_FEAT // D_EDGE

_mesh = plsc.VectorSubcoreMesh(core_axis_name="c", subcore_axis_name="s")


def _zero_rows(buf, n):
    zz = jnp.zeros((16,), jnp.float32)

    def zrow(i, carry):
        for c in range(D_FEAT // 16):
            buf[i, pl.ds(c * 16, 16)] = zz
        return carry

    lax.fori_loop(0, n, zrow, 0)


def _sc_scatter_x(x, src3d, dst3d):
    @functools.partial(
        pl.kernel,
        mesh=_mesh,
        out_type=jax.ShapeDtypeStruct((2, N_PAD, D_FEAT), jnp.float32),
        scratch_types=[
            pltpu.VMEM((2, 1, G), jnp.int32),
            pltpu.VMEM((2, 1, G), jnp.int32),
            pltpu.VMEM((G, D_FEAT), jnp.float32),
            pltpu.VMEM((G, D_FEAT), jnp.float32),
            pltpu.VMEM_SHARED((N_PAD, D_FEAT), jnp.float32),
            pltpu.SemaphoreType.DMA,
            pltpu.SemaphoreType.DMA,
        ],
    )
    def k(x_hbm, src_hbm, dst_hbm, px_hbm, sidx, didx, rows0, rows1,
          acc_x, sem0, sem1):
        cid = lax.axis_index("c")
        sid = lax.axis_index("s")
        wid = sid * 2 + cid

        _zero_rows(rows0, G)
        for i in range(ROWS_PER_TILE // G):
            b = sid * ROWS_PER_TILE + i * G
            pltpu.sync_copy(rows0, acc_x.at[pl.ds(b, G)])
        plsc.subcore_barrier()

        def loadidx(g, b):
            r = wid * GPW + g
            pltpu.sync_copy(src_hbm.at[r], sidx.at[b])
            pltpu.sync_copy(dst_hbm.at[r], didx.at[b])

        def start_gather(b, rows_b, sem_b):
            pltpu.async_copy(x_hbm.at[sidx.at[b].at[0]], rows_b, sem_b)

        def wait_gather(rows_b, sem_b):
            pltpu.make_async_copy(x_hbm.at[pl.ds(0, G)], rows_b, sem_b).wait()

        loadidx(0, 0)
        start_gather(0, rows0, sem0)

        def pair(kk, carry):
            g0 = kk * 2
            loadidx(g0 + 1, 1)
            start_gather(1, rows1, sem1)
            wait_gather(rows0, sem0)
            pltpu.sync_copy(rows0, acc_x.at[didx.at[0].at[0]], add=True)
            loadidx(jnp.minimum(g0 + 2, GPW - 1), 0)
            start_gather(0, rows0, sem0)
            wait_gather(rows1, sem1)
            pltpu.sync_copy(rows1, acc_x.at[didx.at[1].at[0]], add=True)
            return carry

        lax.fori_loop(0, GPW // 2, pair, 0)
        wait_gather(rows0, sem0)
        plsc.subcore_barrier()

        b = sid * ROWS_PER_TILE
        pltpu.sync_copy(acc_x.at[pl.ds(b, ROWS_PER_TILE)],
                        px_hbm.at[cid].at[pl.ds(b, ROWS_PER_TILE)])

    return k(x, src3d, dst3d)


def _sc_scatter_e(ea3d, dst3d):
    @functools.partial(
        pl.kernel,
        mesh=_mesh,
        out_type=jax.ShapeDtypeStruct((2, N_PAD, D_FEAT), jnp.float32),
        scratch_types=[
            pltpu.VMEM((1, G), jnp.int32),
            pltpu.VMEM((D_EDGE, D_FEAT), jnp.float32),
            pltpu.VMEM((G, D_FEAT), jnp.float32),
            pltpu.VMEM_SHARED((N_PAD, D_FEAT), jnp.float32),
        ],
    )
    def k(ea_hbm, dst_hbm, pe_hbm, didx, packed, rows, acc_e):
        cid = lax.axis_index("c")
        sid = lax.axis_index("s")
        wid = sid * 2 + cid

        _zero_rows(rows, G)
        for i in range(ROWS_PER_TILE // G):
            b = sid * ROWS_PER_TILE + i * G
            pltpu.sync_copy(rows, acc_e.at[pl.ds(b, G)])
        plsc.subcore_barrier()

        def group(g, carry):
            r = wid * GPW + g
            pltpu.sync_copy(dst_hbm.at[r], didx)
            rc = jnp.minimum(r, NG - 1)
            pltpu.sync_copy(ea_hbm.at[rc], packed)
            for c in range(PACK):
                for a in range(D_EDGE):
                    rows[a * PACK + c, pl.ds(0, D_EDGE)] = (
                        packed[a, pl.ds(c * D_EDGE, D_EDGE)])
            pltpu.sync_copy(rows, acc_e.at[didx.at[0]], add=True)
            return carry

        lax.fori_loop(0, GPW, group, 0)
        plsc.subcore_barrier()

        b = sid * ROWS_PER_TILE
        pltpu.sync_copy(acc_e.at[pl.ds(b, ROWS_PER_TILE)],
                        pe_hbm.at[cid].at[pl.ds(b, ROWS_PER_TILE)])

    return k(ea3d, dst3d)


def _combine_body(pxa, pxb, pea, peb, o_ref):
    o_ref[:, :D_FEAT] = pxa[0] + pxb[0]
    o_ref[:, D_FEAT:] = pea[0][:, :D_EDGE] + peb[0][:, :D_EDGE]


def _combine(px, pe):
    B = 1000
    return pl.pallas_call(
        _combine_body,
        grid=(N_NODES // B,),
        in_specs=[
            pl.BlockSpec((1, B, D_FEAT), lambda i: (0, i, 0)),
            pl.BlockSpec((1, B, D_FEAT), lambda i: (1, i, 0)),
            pl.BlockSpec((1, B, D_FEAT), lambda i: (0, i, 0)),
            pl.BlockSpec((1, B, D_FEAT), lambda i: (1, i, 0)),
        ],
        out_specs=pl.BlockSpec((B, D_FEAT + D_EDGE), lambda i: (i, 0)),
        out_shape=jax.ShapeDtypeStruct((N_NODES, D_FEAT + D_EDGE), jnp.float32),
    )(px, px, pe, pe)


def kernel(x, edge_index, edge_attr):
    src = edge_index[0].astype(jnp.int32)
    dst = edge_index[1].astype(jnp.int32)
    pad = E_PAD - N_EDGES
    pad_ids = jnp.arange(pad, dtype=jnp.int32)
    src3d = jnp.concatenate(
        [src, pad_ids % N_NODES]).reshape(NG_PAD, 1, G)
    dst3d = jnp.concatenate(
        [dst, N_NODES + pad_ids % (N_PAD - N_NODES)]).reshape(NG_PAD, 1, G)
    ea3d = edge_attr.reshape(NG, D_EDGE, D_FEAT)
    px = _sc_scatter_x(x, src3d, dst3d)
    pe = _sc_scatter_e(ea3d, dst3d)
    return _combine(px, pe)


# speedup vs baseline: 7.3496x; 1.2263x over previous
import functools

import jax
import jax.numpy as jnp
from jax import lax
from jax.experimental import pallas as pl
from jax.experimental.pallas import tpu as pltpu
from jax.experimental.pallas import tpu_sc as plsc

N_NODES = 10000
N_EDGES = 320000
D_FEAT = 128
D_EDGE = 16
G = 128
GPW = 80
N_WORKERS = 32
NG = N_EDGES // G
NG_PAD = N_WORKERS * GPW
E_PAD = NG_PAD * G
N_TILES = 16
N_PAD = 10240
ROWS_PER_TILE = N_PAD // N_TILES
PACK = D_FEAT // D_EDGE

_mesh = plsc.VectorSubcoreMesh(core_axis_name="c", subcore_axis_name="s")


def _zero_rows(buf, n):
    zz = jnp.zeros((16,), jnp.float32)

    def zrow(i, carry):
        for c in range(D_FEAT // 16):
            buf[i, pl.ds(c * 16, 16)] = zz
        return carry

    lax.fori_loop(0, n, zrow, 0)


def _sc_scatter_x(x, src3d, dst3d):
    @functools.partial(
        pl.kernel,
        mesh=_mesh,
        out_type=jax.ShapeDtypeStruct((2, N_PAD, D_FEAT), jnp.float32),
        scratch_types=[
            pltpu.VMEM((GPW // 2, G), jnp.int32),
            pltpu.VMEM((GPW // 2, G), jnp.int32),
            pltpu.VMEM((G, D_FEAT), jnp.float32),
            pltpu.VMEM((G, D_FEAT), jnp.float32),
            pltpu.VMEM_SHARED((N_PAD, D_FEAT), jnp.float32),
            pltpu.SemaphoreType.DMA,
            pltpu.SemaphoreType.DMA,
        ],
    )
    def k(x_hbm, src_hbm, dst_hbm, px_hbm, sidx, didx, rows0, rows1,
          acc_x, sem0, sem1):
        cid = lax.axis_index("c")
        sid = lax.axis_index("s")
        wid = sid * 2 + cid
        H = GPW // 2

        _zero_rows(rows0, G)
        for i in range(ROWS_PER_TILE // G):
            b = sid * ROWS_PER_TILE + i * G
            pltpu.sync_copy(rows0, acc_x.at[pl.ds(b, G)])
        plsc.subcore_barrier()

        def start_gather(j, rows_b, sem_b):
            pltpu.async_copy(x_hbm.at[sidx.at[j]], rows_b, sem_b)

        def wait_gather(rows_b, sem_b):
            pltpu.make_async_copy(x_hbm.at[pl.ds(0, G)], rows_b, sem_b).wait()

        for h in range(2):
            slab = wid * 2 + h
            pltpu.sync_copy(src_hbm.at[slab], sidx)
            pltpu.sync_copy(dst_hbm.at[slab], didx)
            start_gather(0, rows0, sem0)

            def pair(kk, carry):
                j0 = kk * 2
                start_gather(j0 + 1, rows1, sem1)
                wait_gather(rows0, sem0)
                pltpu.sync_copy(rows0, acc_x.at[didx.at[j0]], add=True)
                start_gather(jnp.minimum(j0 + 2, H - 1), rows0, sem0)
                wait_gather(rows1, sem1)
                pltpu.sync_copy(rows1, acc_x.at[didx.at[j0 + 1]], add=True)
                return carry

            lax.fori_loop(0, H // 2, pair, 0)
            wait_gather(rows0, sem0)
        plsc.subcore_barrier()

        b = sid * ROWS_PER_TILE
        pltpu.sync_copy(acc_x.at[pl.ds(b, ROWS_PER_TILE)],
                        px_hbm.at[cid].at[pl.ds(b, ROWS_PER_TILE)])

    return k(x, src3d, dst3d)


def _sc_scatter_e(ea3d, dst3d):
    @functools.partial(
        pl.kernel,
        mesh=_mesh,
        out_type=jax.ShapeDtypeStruct((2, N_PAD, D_FEAT), jnp.float32),
        scratch_types=[
            pltpu.VMEM((GPW // 2, G), jnp.int32),
            pltpu.VMEM((D_EDGE, D_FEAT), jnp.float32),
            pltpu.VMEM((G, D_FEAT), jnp.float32),
            pltpu.VMEM_SHARED((N_PAD, D_FEAT), jnp.float32),
        ],
    )
    def k(ea_hbm, dst_hbm, pe_hbm, didx, packed, rows, acc_e):
        cid = lax.axis_index("c")
        sid = lax.axis_index("s")
        wid = sid * 2 + cid
        H = GPW // 2

        _zero_rows(rows, G)
        for i in range(ROWS_PER_TILE // G):
            b = sid * ROWS_PER_TILE + i * G
            pltpu.sync_copy(rows, acc_e.at[pl.ds(b, G)])
        plsc.subcore_barrier()

        for h in range(2):
            slab = wid * 2 + h
            pltpu.sync_copy(dst_hbm.at[slab], didx)

            def group(j, carry):
                r = wid * GPW + h * H + j
                rc = jnp.minimum(r, NG - 1)
                pltpu.sync_copy(ea_hbm.at[rc], packed)
                for c in range(PACK):
                    for a in range(D_EDGE):
                        rows[a * PACK + c, pl.ds(0, D_EDGE)] = (
                            packed[a, pl.ds(c * D_EDGE, D_EDGE)])
                pltpu.sync_copy(rows, acc_e.at[didx.at[j]], add=True)
                return carry

            lax.fori_loop(0, H, group, 0)
        plsc.subcore_barrier()

        b = sid * ROWS_PER_TILE
        pltpu.sync_copy(acc_e.at[pl.ds(b, ROWS_PER_TILE)],
                        pe_hbm.at[cid].at[pl.ds(b, ROWS_PER_TILE)])

    return k(ea3d, dst3d)


def _combine_body(pxa, pxb, pea, peb, o_ref):
    o_ref[:, :D_FEAT] = pxa[0] + pxb[0]
    o_ref[:, D_FEAT:] = pea[0][:, :D_EDGE] + peb[0][:, :D_EDGE]


def _combine(px, pe):
    B = 1000
    return pl.pallas_call(
        _combine_body,
        grid=(N_NODES // B,),
        in_specs=[
            pl.BlockSpec((1, B, D_FEAT), lambda i: (0, i, 0)),
            pl.BlockSpec((1, B, D_FEAT), lambda i: (1, i, 0)),
            pl.BlockSpec((1, B, D_FEAT), lambda i: (0, i, 0)),
            pl.BlockSpec((1, B, D_FEAT), lambda i: (1, i, 0)),
        ],
        out_specs=pl.BlockSpec((B, D_FEAT + D_EDGE), lambda i: (i, 0)),
        out_shape=jax.ShapeDtypeStruct((N_NODES, D_FEAT + D_EDGE), jnp.float32),
    )(px, px, pe, pe)


def kernel(x, edge_index, edge_attr):
    src = edge_index[0].astype(jnp.int32)
    dst = edge_index[1].astype(jnp.int32)
    pad = E_PAD - N_EDGES
    pad_ids = jnp.arange(pad, dtype=jnp.int32)
    src3d = jnp.concatenate(
        [src, pad_ids % N_NODES]).reshape(N_WORKERS * 2, GPW // 2, G)
    dst3d = jnp.concatenate(
        [dst, N_NODES + pad_ids % (N_PAD - N_NODES)]).reshape(
            N_WORKERS * 2, GPW // 2, G)
    ea3d = edge_attr.reshape(NG, D_EDGE, D_FEAT)
    px = _sc_scatter_x(x, src3d, dst3d)
    pe = _sc_scatter_e(ea3d, dst3d)
    return _combine(px, pe)


# speedup vs baseline: 8.5204x; 1.1593x over previous
import functools

import jax
import jax.numpy as jnp
from jax import lax
from jax.experimental import pallas as pl
from jax.experimental.pallas import tpu as pltpu
from jax.experimental.pallas import tpu_sc as plsc

N_NODES = 10000
N_EDGES = 320000
D_FEAT = 128
D_EDGE = 16
G = 128
GPW = 80
N_WORKERS = 32
NG = N_EDGES // G
NG_PAD = N_WORKERS * GPW
E_PAD = NG_PAD * G
N_TILES = 16
N_PAD = 10240
ROWS_PER_TILE = N_PAD // N_TILES
PACK = D_FEAT // D_EDGE

_mesh = plsc.VectorSubcoreMesh(core_axis_name="c", subcore_axis_name="s")


def _zero_rows(buf, n):
    zz = jnp.zeros((16,), jnp.float32)

    def zrow(i, carry):
        for c in range(D_FEAT // 16):
            buf[i, pl.ds(c * 16, 16)] = zz
        return carry

    lax.fori_loop(0, n, zrow, 0)


def _sc_scatter_x(x, src3d, dst3d):
    @functools.partial(
        pl.kernel,
        mesh=_mesh,
        out_type=jax.ShapeDtypeStruct((2, N_PAD, D_FEAT), jnp.float32),
        scratch_types=[
            pltpu.VMEM((GPW // 2, G), jnp.int32),
            pltpu.VMEM((GPW // 2, G), jnp.int32),
            pltpu.VMEM((G, D_FEAT), jnp.float32),
            pltpu.VMEM((G, D_FEAT), jnp.float32),
            pltpu.VMEM_SHARED((N_PAD, D_FEAT), jnp.float32),
            pltpu.SemaphoreType.DMA,
            pltpu.SemaphoreType.DMA,
        ],
    )
    def k(x_hbm, src_hbm, dst_hbm, px_hbm, sidx, didx, rows0, rows1,
          acc_x, sem0, sem1):
        cid = lax.axis_index("c")
        sid = lax.axis_index("s")
        wid = sid * 2 + cid
        H = GPW // 2

        _zero_rows(rows0, G)
        for i in range(ROWS_PER_TILE // G):
            b = sid * ROWS_PER_TILE + i * G
            pltpu.sync_copy(rows0, acc_x.at[pl.ds(b, G)])
        plsc.subcore_barrier()

        def start_gather(j, rows_b, sem_b):
            pltpu.async_copy(x_hbm.at[sidx.at[j]], rows_b, sem_b)

        def wait_gather(rows_b, sem_b):
            pltpu.make_async_copy(x_hbm.at[pl.ds(0, G)], rows_b, sem_b).wait()

        for h in range(2):
            slab = wid * 2 + h
            pltpu.sync_copy(src_hbm.at[slab], sidx)
            pltpu.sync_copy(dst_hbm.at[slab], didx)
            start_gather(0, rows0, sem0)

            def pair(kk, carry):
                j0 = kk * 2
                start_gather(j0 + 1, rows1, sem1)
                wait_gather(rows0, sem0)
                pltpu.sync_copy(rows0, acc_x.at[didx.at[j0]], add=True)
                start_gather(jnp.minimum(j0 + 2, H - 1), rows0, sem0)
                wait_gather(rows1, sem1)
                pltpu.sync_copy(rows1, acc_x.at[didx.at[j0 + 1]], add=True)
                return carry

            lax.fori_loop(0, H // 2, pair, 0)
            wait_gather(rows0, sem0)
        plsc.subcore_barrier()

        b = sid * ROWS_PER_TILE
        pltpu.sync_copy(acc_x.at[pl.ds(b, ROWS_PER_TILE)],
                        px_hbm.at[cid].at[pl.ds(b, ROWS_PER_TILE)])

    return k(x, src3d, dst3d)


def _sc_scatter_e(ea3d, dst3d):
    @functools.partial(
        pl.kernel,
        mesh=_mesh,
        out_type=jax.ShapeDtypeStruct((2, N_PAD, D_FEAT), jnp.float32),
        scratch_types=[
            pltpu.VMEM((GPW // 2, G), jnp.int32),
            pltpu.VMEM((D_EDGE, D_FEAT), jnp.float32),
            pltpu.VMEM((D_EDGE, D_FEAT), jnp.float32),
            pltpu.VMEM((G, D_FEAT), jnp.float32),
            pltpu.VMEM((G, D_FEAT), jnp.float32),
            pltpu.VMEM_SHARED((N_PAD, D_FEAT), jnp.float32),
            pltpu.SemaphoreType.DMA,
            pltpu.SemaphoreType.DMA,
        ],
    )
    def k(ea_hbm, dst_hbm, pe_hbm, didx, pk0, pk1, er0, er1, acc_e,
          sem0, sem1):
        cid = lax.axis_index("c")
        sid = lax.axis_index("s")
        wid = sid * 2 + cid
        H = GPW // 2

        _zero_rows(er0, G)
        _zero_rows(er1, G)
        for i in range(ROWS_PER_TILE // G):
            b = sid * ROWS_PER_TILE + i * G
            pltpu.sync_copy(er0, acc_e.at[pl.ds(b, G)])
        plsc.subcore_barrier()

        def load_packed(h, j, pk):
            r = wid * GPW + h * H + j
            pltpu.sync_copy(ea_hbm.at[jnp.minimum(r, NG - 1)], pk)

        def unpack(pk, er):
            for c in range(PACK):
                for a in range(D_EDGE):
                    er[a * PACK + c, pl.ds(0, D_EDGE)] = (
                        pk[a, pl.ds(c * D_EDGE, D_EDGE)])

        for h in range(2):
            slab = wid * 2 + h
            pltpu.sync_copy(dst_hbm.at[slab], didx)
            load_packed(h, 0, pk0)
            unpack(pk0, er0)

            def pair(kk, carry):
                j0 = kk * 2
                d0 = pltpu.async_copy(er0, acc_e.at[didx.at[j0]], sem0,
                                      add=True)
                load_packed(h, j0 + 1, pk1)
                unpack(pk1, er1)
                d0.wait()
                d1 = pltpu.async_copy(er1, acc_e.at[didx.at[j0 + 1]], sem1,
                                      add=True)
                load_packed(h, jnp.minimum(j0 + 2, H - 1), pk0)
                unpack(pk0, er0)
                d1.wait()
                return carry

            lax.fori_loop(0, H // 2, pair, 0)
        plsc.subcore_barrier()

        b = sid * ROWS_PER_TILE
        pltpu.sync_copy(acc_e.at[pl.ds(b, ROWS_PER_TILE)],
                        pe_hbm.at[cid].at[pl.ds(b, ROWS_PER_TILE)])

    return k(ea3d, dst3d)


def _combine_body(pxa, pxb, pea, peb, o_ref):
    o_ref[:, :D_FEAT] = pxa[0] + pxb[0]
    o_ref[:, D_FEAT:] = pea[0][:, :D_EDGE] + peb[0][:, :D_EDGE]


def _combine(px, pe):
    B = 1000
    return pl.pallas_call(
        _combine_body,
        grid=(N_NODES // B,),
        in_specs=[
            pl.BlockSpec((1, B, D_FEAT), lambda i: (0, i, 0)),
            pl.BlockSpec((1, B, D_FEAT), lambda i: (1, i, 0)),
            pl.BlockSpec((1, B, D_FEAT), lambda i: (0, i, 0)),
            pl.BlockSpec((1, B, D_FEAT), lambda i: (1, i, 0)),
        ],
        out_specs=pl.BlockSpec((B, D_FEAT + D_EDGE), lambda i: (i, 0)),
        out_shape=jax.ShapeDtypeStruct((N_NODES, D_FEAT + D_EDGE), jnp.float32),
    )(px, px, pe, pe)


def kernel(x, edge_index, edge_attr):
    src = edge_index[0].astype(jnp.int32)
    dst = edge_index[1].astype(jnp.int32)
    pad = E_PAD - N_EDGES
    pad_ids = jnp.arange(pad, dtype=jnp.int32)
    src3d = jnp.concatenate(
        [src, pad_ids % N_NODES]).reshape(N_WORKERS * 2, GPW // 2, G)
    dst3d = jnp.concatenate(
        [dst, N_NODES + pad_ids % (N_PAD - N_NODES)]).reshape(
            N_WORKERS * 2, GPW // 2, G)
    ea3d = edge_attr.reshape(NG, D_EDGE, D_FEAT)
    px = _sc_scatter_x(x, src3d, dst3d)
    pe = _sc_scatter_e(ea3d, dst3d)
    return _combine(px, pe)
